# baseline (speedup 1.0000x reference)
import functools

import jax
import jax.numpy as jnp
from jax import lax
from jax.experimental import pallas as pl
from jax.experimental.pallas import tpu as pltpu
from jax.experimental.pallas import tpu_sc as plsc

NUM_AXIOMS = 4096
D_AXIOM = 449
D_PAD = 464
BATCH = 16384

_NUM_CORES = 2
_NUM_SUBCORES = 16
_NW = _NUM_CORES * _NUM_SUBCORES
_B_PER_W = BATCH // _NW
_CHUNK = 64
_NCHUNK = _B_PER_W // _CHUNK
_NVEC = D_PAD // 16
_FLAT = _CHUNK * D_AXIOM
_ROWS_PER_SUB = NUM_AXIOMS // _NUM_SUBCORES
_NSTAGE = _ROWS_PER_SUB // _CHUNK

_mesh = plsc.VectorSubcoreMesh(core_axis_name="c", subcore_axis_name="s")


def _expand(srcflat, dst2d):
    iota = lax.broadcasted_iota(jnp.int32, (16,), 0)

    def row_fn(r, carry):
        drow = dst2d.at[r]
        sbase = r * D_AXIOM
        for k in range(_NVEC):
            v = plsc.load_gather(srcflat, [sbase + (k * 16) + iota])
            drow[pl.ds(k * 16, 16)] = v
        return carry

    lax.fori_loop(0, _CHUNK, row_fn, 0)


def _compact(src2d, dstflat):
    iota = lax.broadcasted_iota(jnp.int32, (16,), 0)

    def row_fn(r, carry):
        srow = src2d.at[r]
        dbase = r * D_AXIOM
        for k in range(_NVEC):
            v = srow[pl.ds(k * 16, 16)]
            plsc.store_scatter(dstflat, [dbase + (k * 16) + iota], v)
        return carry

    lax.fori_loop(0, _CHUNK, row_fn, 0)


@functools.partial(
    pl.kernel,
    mesh=_mesh,
    out_type=jax.ShapeDtypeStruct((BATCH * D_AXIOM,), jnp.float32),
    compiler_params=pltpu.CompilerParams(
        use_tc_tiling_on_sc=False, needs_layout_passes=False
    ),
    scratch_types=[
        pltpu.VMEM((_NCHUNK, _CHUNK), jnp.int32),
        pltpu.VMEM((_CHUNK, D_PAD), jnp.float32),
        pltpu.VMEM((_CHUNK, D_PAD), jnp.float32),
        pltpu.VMEM((_FLAT + 16,), jnp.float32),
        pltpu.VMEM((_FLAT + 16,), jnp.float32),
        pltpu.HBM((NUM_AXIOMS, D_PAD), jnp.float32),
        pltpu.SemaphoreType.DMA,
        pltpu.SemaphoreType.DMA,
        pltpu.SemaphoreType.DMA,
        pltpu.SemaphoreType.DMA,
    ],
)
def _gather_kernel(idx_hbm, table_hbm, out_hbm,
                   idx_v, rows0, rows1, flat0, flat1, padded_hbm,
                   sg0, sg1, sw0, sw1):
    cid = lax.axis_index("c")
    sid = lax.axis_index("s")
    wid = sid * _NUM_CORES + cid
    base = wid * _B_PER_W
    rows = (rows0, rows1)
    flats = (flat0, flat1)
    sgs = (sg0, sg1)
    sws = (sw0, sw1)

    pltpu.sync_copy(idx_hbm.at[wid], idx_v)

    row0 = sid * _ROWS_PER_SUB
    scp = [
        pltpu.async_copy(
            table_hbm.at[pl.ds(row0 * D_AXIOM, _FLAT)],
            flat0.at[pl.ds(0, _FLAT)], sg0,
        ),
        None,
    ]
    for q in range(_NSTAGE):
        cur = q % 2
        nxt = (q + 1) % 2
        if q + 1 < _NSTAGE:
            scp[nxt] = pltpu.async_copy(
                table_hbm.at[pl.ds((row0 + (q + 1) * _CHUNK) * D_AXIOM, _FLAT)],
                flats[nxt].at[pl.ds(0, _FLAT)], sgs[nxt],
            )
        scp[cur].wait()
        _expand(flats[cur], rows[cur])
        pltpu.sync_copy(
            rows[cur], padded_hbm.at[pl.ds(row0 + q * _CHUNK, _CHUNK)]
        )
    plsc.subcore_barrier()

    gcp = [pltpu.async_copy(padded_hbm.at[idx_v.at[0]], rows0, sg0), None]
    wcp = [None, None]
    for j in range(_NCHUNK):
        cur = j % 2
        nxt = (j + 1) % 2
        if j + 1 < _NCHUNK:
            gcp[nxt] = pltpu.async_copy(
                padded_hbm.at[idx_v.at[j + 1]], rows[nxt], sgs[nxt]
            )
        gcp[cur].wait()
        if wcp[cur] is not None:
            wcp[cur].wait()
        _compact(rows[cur], flats[cur])
        wcp[cur] = pltpu.async_copy(
            flats[cur].at[pl.ds(0, _FLAT)],
            out_hbm.at[pl.ds((base + j * _CHUNK) * D_AXIOM, _FLAT)],
            sws[cur],
        )
    wcp[0].wait()
    wcp[1].wait()


def kernel(indices, table):
    idx = indices.astype(jnp.int32).reshape(_NW, _NCHUNK, _CHUNK)
    out_flat = _gather_kernel(idx, table.reshape(-1))
    return out_flat.reshape(BATCH, D_AXIOM)


# speedup vs baseline: 1.3714x; 1.3714x over previous
import functools

import jax
import jax.numpy as jnp
from jax import lax
from jax.experimental import pallas as pl
from jax.experimental.pallas import tpu as pltpu
from jax.experimental.pallas import tpu_sc as plsc

NUM_AXIOMS = 4096
D_AXIOM = 449
D_PAD = 464
BATCH = 16384

_NUM_CORES = 2
_NUM_SUBCORES = 16
_NW = _NUM_CORES * _NUM_SUBCORES
_B_PER_W = BATCH // _NW
_CHUNK = 64
_NCHUNK = _B_PER_W // _CHUNK
_NVEC = D_PAD // 16

_mesh = plsc.VectorSubcoreMesh(core_axis_name="c", subcore_axis_name="s")


def _compact(src2d, dst2d):
    iota = lax.broadcasted_iota(jnp.int32, (16,), 0)
    tail_mask = iota < (D_AXIOM - (_NVEC - 1) * 16)

    def body(r):
        row_ids = iota * 0 + r
        srow = src2d.at[r]
        for k in range(_NVEC - 1):
            v = srow[pl.ds(k * 16, 16)]
            plsc.store_scatter(dst2d, [row_ids, (k * 16) + iota], v)
        v = srow[pl.ds((_NVEC - 1) * 16, 16)]
        plsc.store_scatter(
            dst2d, [row_ids, ((_NVEC - 1) * 16) + iota], v, mask=tail_mask
        )

    plsc.parallel_loop(0, _CHUNK, unroll=2)(body)


@functools.partial(
    pl.kernel,
    mesh=_mesh,
    out_type=jax.ShapeDtypeStruct((BATCH, D_AXIOM), jnp.float32),
    compiler_params=pltpu.CompilerParams(
        use_tc_tiling_on_sc=False, needs_layout_passes=False
    ),
    scratch_types=[
        pltpu.VMEM((_NCHUNK, _CHUNK), jnp.int32),
        pltpu.VMEM((_CHUNK, D_PAD), jnp.float32),
        pltpu.VMEM((_CHUNK, D_PAD), jnp.float32),
        pltpu.VMEM((_CHUNK, D_AXIOM), jnp.float32),
        pltpu.VMEM((_CHUNK, D_AXIOM), jnp.float32),
        pltpu.SemaphoreType.DMA,
        pltpu.SemaphoreType.DMA,
        pltpu.SemaphoreType.DMA,
        pltpu.SemaphoreType.DMA,
    ],
)
def _gather_kernel(idx_hbm, table_hbm, out_hbm,
                   idx_v, rows0, rows1, dense0, dense1, sg0, sg1, sw0, sw1):
    wid = lax.axis_index("s") * _NUM_CORES + lax.axis_index("c")
    base = wid * _B_PER_W
    pltpu.sync_copy(idx_hbm.at[wid], idx_v)
    rows = (rows0, rows1)
    denses = (dense0, dense1)
    sgs = (sg0, sg1)
    sws = (sw0, sw1)
    gcp = [pltpu.async_copy(table_hbm.at[idx_v.at[0]], rows0, sg0), None]
    wcp = [None, None]
    for j in range(_NCHUNK):
        cur = j % 2
        nxt = (j + 1) % 2
        if j + 1 < _NCHUNK:
            gcp[nxt] = pltpu.async_copy(
                table_hbm.at[idx_v.at[j + 1]], rows[nxt], sgs[nxt]
            )
        gcp[cur].wait()
        if wcp[cur] is not None:
            wcp[cur].wait()
        _compact(rows[cur], denses[cur])
        wcp[cur] = pltpu.async_copy(
            denses[cur],
            out_hbm.at[pl.ds(base + j * _CHUNK, _CHUNK)],
            sws[cur],
        )
    wcp[0].wait()
    wcp[1].wait()


def kernel(indices, table):
    idx = indices.astype(jnp.int32).reshape(_NW, _NCHUNK, _CHUNK)
    table_pad = jnp.pad(table, ((0, 0), (0, D_PAD - D_AXIOM)))
    return _gather_kernel(idx, table_pad)
